# baseline (speedup 1.0000x reference)
import jax
import jax.numpy as jnp
from jax import lax
from jax.experimental import pallas as pl
from jax.experimental.pallas import tpu as pltpu
from jax.experimental.pallas import tpu_sc as plsc

B = 16384
D = 128
NC = 2
NS = 16
NW = NC * NS
BPW = B // NW
CH = 128
NCH = BPW // CH
GROUPS = CH // 16


def _mirt_body(uids_hbm, iids_hbm, emb_hbm, disc_hbm, diff_hbm, out_hbm,
               uid_v, iid_v, theta_v, a_v, b_v, out_v, pscr_v,
               sem0, sem1, sem_out):
    wid = lax.axis_index("s") * NC + lax.axis_index("c")
    base = wid * BPW

    cp_u = pltpu.async_copy(uids_hbm.at[pl.ds(base, BPW)], uid_v, sem0)
    cp_i = pltpu.async_copy(iids_hbm.at[pl.ds(base, BPW)], iid_v, sem1)
    cp_u.wait()
    cp_i.wait()

    iota = lax.iota(jnp.int32, 16)
    sems = (sem0, sem1)

    def start(c):
        p = c % 2
        ids = pl.ds(c * CH, CH)
        return (
            pltpu.async_copy(emb_hbm.at[uid_v.at[ids]], theta_v.at[p], sems[p]),
            pltpu.async_copy(disc_hbm.at[iid_v.at[ids]], a_v.at[p], sems[p]),
            pltpu.async_copy(diff_hbm.at[iid_v.at[ids]], b_v.at[p], sems[p]),
        )

    pending = start(0)
    out_pending = []
    for c in range(NCH):
        p = c % 2
        for cp in pending:
            cp.wait()
        if c + 1 < NCH:
            pending = start(c + 1)

        def one_group(g, pbase, p=p, c=c):
            for eb in range(0, 16, 4):
                accs = []
                for e16 in range(eb, eb + 4):
                    e = g * 16 + e16
                    acc = None
                    for j in range(D // 16):
                        tv = theta_v[p, e, pl.ds(j * 16, 16)]
                        av = a_v[p, e, pl.ds(j * 16, 16)]
                        prod = tv * av
                        acc = prod if acc is None else acc + prod
                    accs.append(acc)
                for k, acc in enumerate(accs):
                    pscr_v[pl.ds(pbase + (eb + k) * 17, 16)] = acc
            idx = pbase + iota * 17
            out16 = None
            for j in range(16):
                col = plsc.load_gather(pscr_v, [idx])
                out16 = col if out16 is None else out16 + col
                idx = idx + 1
            b16 = b_v[p, pl.ds(g * 16, 16)]
            x = out16 - b16
            out_v[pl.ds(c * CH + g * 16, 16)] = 1.0 / (1.0 + jnp.exp(-x))

        def pair_body(i, _):
            one_group(2 * i, 0)
            one_group(2 * i + 1, 16 * 17)
            return 0

        lax.fori_loop(0, GROUPS // 2, pair_body, 0)

        out_pending.append(pltpu.async_copy(
            out_v.at[pl.ds(c * CH, CH)],
            out_hbm.at[pl.ds(base + c * CH, CH)], sem_out))

    for cp in out_pending:
        cp.wait()


@jax.jit
def _mirt_sc(uids, iids, users_emb, item_discrimination, diff):
    mesh = plsc.VectorSubcoreMesh(core_axis_name="c", subcore_axis_name="s")
    run = pl.kernel(
        _mirt_body,
        out_type=jax.ShapeDtypeStruct((B,), jnp.float32),
        mesh=mesh,
        scratch_types=[
            pltpu.VMEM((BPW,), jnp.int32),
            pltpu.VMEM((BPW,), jnp.int32),
            pltpu.VMEM((2, CH, D), jnp.float32),
            pltpu.VMEM((2, CH, D), jnp.float32),
            pltpu.VMEM((2, CH), jnp.float32),
            pltpu.VMEM((BPW,), jnp.float32),
            pltpu.VMEM((2 * 16 * 17,), jnp.float32),
            pltpu.SemaphoreType.DMA,
            pltpu.SemaphoreType.DMA,
            pltpu.SemaphoreType.DMA,
        ],
        compiler_params=pltpu.CompilerParams(needs_layout_passes=False),
    )
    return run(uids, iids, users_emb, item_discrimination, diff)


def kernel(user_ids, item_ids, concept_ids, users_emb, item_discrimination,
           item_difficulty):
    del concept_ids
    return _mirt_sc(user_ids, item_ids, users_emb, item_discrimination,
                    item_difficulty.reshape(-1))


# speedup vs baseline: 1.0818x; 1.0818x over previous
import jax
import jax.numpy as jnp
from jax import lax
from jax.experimental import pallas as pl
from jax.experimental.pallas import tpu as pltpu
from jax.experimental.pallas import tpu_sc as plsc

B = 16384
D = 128
NC = 2
NS = 16
NW = NC * NS
BPW = B // NW
CH = 128
NCH = BPW // CH
GROUPS = CH // 16


def _mirt_body(uids_hbm, iids_hbm, emb_hbm, disc_hbm, diff_hbm, out_hbm,
               uid_v, iid_v, theta_v, a_v, b_v, out_v, pscr_v,
               sem0, sem1, sem_out):
    wid = lax.axis_index("s") * NC + lax.axis_index("c")
    base = wid * BPW

    iota = lax.iota(jnp.int32, 16)
    sems = (sem0, sem1)

    def start(c):
        p = c % 2
        ids = pl.ds(c * CH, CH)
        return (
            pltpu.async_copy(emb_hbm.at[uid_v.at[ids]], theta_v.at[p], sems[p]),
            pltpu.async_copy(disc_hbm.at[iid_v.at[ids]], a_v.at[p], sems[p]),
            pltpu.async_copy(diff_hbm.at[iid_v.at[ids]], b_v.at[p], sems[p]),
        )

    cp_u = pltpu.async_copy(uids_hbm.at[pl.ds(base, BPW)], uid_v, sem0)
    cp_i = pltpu.async_copy(iids_hbm.at[pl.ds(base, BPW)], iid_v, sem1)
    cp_u.wait()
    ids0 = pl.ds(0, CH)
    t0 = pltpu.async_copy(emb_hbm.at[uid_v.at[ids0]], theta_v.at[0], sems[0])
    cp_i.wait()
    a0 = pltpu.async_copy(disc_hbm.at[iid_v.at[ids0]], a_v.at[0], sems[0])
    b0 = pltpu.async_copy(diff_hbm.at[iid_v.at[ids0]], b_v.at[0], sems[0])

    pending = (t0, a0, b0)
    out_pending = []
    for c in range(NCH):
        p = c % 2
        nxt = start(c + 1) if c + 1 < NCH else ()
        for cp in pending:
            cp.wait()
        pending = nxt

        def group_body(g, _, p=p, c=c):
            for eb in range(0, 16, 4):
                accs = []
                for e16 in range(eb, eb + 4):
                    e = g * 16 + e16
                    acc = None
                    for j in range(D // 16):
                        tv = theta_v[p, e, pl.ds(j * 16, 16)]
                        av = a_v[p, e, pl.ds(j * 16, 16)]
                        prod = tv * av
                        acc = prod if acc is None else acc + prod
                    accs.append(acc)
                for k, acc in enumerate(accs):
                    pscr_v[pl.ds((eb + k) * 17, 16)] = acc
            idx = iota * 17
            out16 = None
            for j in range(16):
                col = plsc.load_gather(pscr_v, [idx])
                out16 = col if out16 is None else out16 + col
                idx = idx + 1
            b16 = b_v[p, pl.ds(g * 16, 16)]
            x = out16 - b16
            out_v[pl.ds(c * CH + g * 16, 16)] = 1.0 / (1.0 + jnp.exp(-x))
            return 0

        lax.fori_loop(0, GROUPS, group_body, 0)

        out_pending.append(pltpu.async_copy(
            out_v.at[pl.ds(c * CH, CH)],
            out_hbm.at[pl.ds(base + c * CH, CH)], sem_out))

    for cp in out_pending:
        cp.wait()


@jax.jit
def _mirt_sc(uids, iids, users_emb, item_discrimination, diff):
    mesh = plsc.VectorSubcoreMesh(core_axis_name="c", subcore_axis_name="s")
    run = pl.kernel(
        _mirt_body,
        out_type=jax.ShapeDtypeStruct((B,), jnp.float32),
        mesh=mesh,
        scratch_types=[
            pltpu.VMEM((BPW,), jnp.int32),
            pltpu.VMEM((BPW,), jnp.int32),
            pltpu.VMEM((2, CH, D), jnp.float32),
            pltpu.VMEM((2, CH, D), jnp.float32),
            pltpu.VMEM((2, CH), jnp.float32),
            pltpu.VMEM((BPW,), jnp.float32),
            pltpu.VMEM((16 * 17,), jnp.float32),
            pltpu.SemaphoreType.DMA,
            pltpu.SemaphoreType.DMA,
            pltpu.SemaphoreType.DMA,
        ],
        compiler_params=pltpu.CompilerParams(needs_layout_passes=False),
    )
    return run(uids, iids, users_emb, item_discrimination, diff)


def kernel(user_ids, item_ids, concept_ids, users_emb, item_discrimination,
           item_difficulty):
    del concept_ids
    return _mirt_sc(user_ids, item_ids, users_emb, item_discrimination,
                    item_difficulty.reshape(-1))


# speedup vs baseline: 1.0976x; 1.0146x over previous
import jax
import jax.numpy as jnp
from jax import lax
from jax.experimental import pallas as pl
from jax.experimental.pallas import tpu as pltpu
from jax.experimental.pallas import tpu_sc as plsc

B = 16384
D = 128
NC = 2
NS = 16
NW = NC * NS
BPW = B // NW
CH = 128
NCH = BPW // CH
GROUPS = CH // 16


def _mirt_body(uids_hbm, iids_hbm, emb_hbm, disc_hbm, diff_hbm, out_hbm,
               uid_v, iid_v, theta_v, a_v, b_v, out_v, pscr_v,
               sem0, sem1, sem_out):
    wid = lax.axis_index("s") * NC + lax.axis_index("c")
    base = wid * BPW

    cp_u = pltpu.async_copy(uids_hbm.at[pl.ds(base, BPW)], uid_v, sem0)
    cp_i = pltpu.async_copy(iids_hbm.at[pl.ds(base, BPW)], iid_v, sem1)
    cp_u.wait()
    cp_i.wait()

    iota = lax.iota(jnp.int32, 16)
    sems = (sem0, sem1)

    def start(c):
        p = c % 2
        ids = pl.ds(c * CH, CH)
        return (
            pltpu.async_copy(emb_hbm.at[uid_v.at[ids]], theta_v.at[p], sems[p]),
            pltpu.async_copy(disc_hbm.at[iid_v.at[ids]], a_v.at[p], sems[p]),
            pltpu.async_copy(diff_hbm.at[iid_v.at[ids]], b_v.at[p], sems[p]),
        )

    pending = start(0)
    out_pending = []
    for c in range(NCH):
        p = c % 2
        for cp in pending:
            cp.wait()
        if c + 1 < NCH:
            pending = start(c + 1)

        def group_body(g, _, p=p, c=c):
            for eb in range(0, 16, 4):
                accs = []
                for e16 in range(eb, eb + 4):
                    e = g * 16 + e16
                    acc = None
                    for j in range(D // 16):
                        tv = theta_v[p, e, pl.ds(j * 16, 16)]
                        av = a_v[p, e, pl.ds(j * 16, 16)]
                        prod = tv * av
                        acc = prod if acc is None else acc + prod
                    accs.append(acc)
                for k, acc in enumerate(accs):
                    pscr_v[pl.ds((eb + k) * 17, 16)] = acc
            idx = iota * 17
            out16 = None
            for j in range(16):
                col = plsc.load_gather(pscr_v, [idx])
                out16 = col if out16 is None else out16 + col
                idx = idx + 1
            b16 = b_v[p, pl.ds(g * 16, 16)]
            x = out16 - b16
            out_v[pl.ds(c * CH + g * 16, 16)] = 1.0 / (1.0 + jnp.exp(-x))
            return 0

        lax.fori_loop(0, GROUPS, group_body, 0)

        out_pending.append(pltpu.async_copy(
            out_v.at[pl.ds(c * CH, CH)],
            out_hbm.at[pl.ds(base + c * CH, CH)], sem_out))

    for cp in out_pending:
        cp.wait()


@jax.jit
def _mirt_sc(uids, iids, users_emb, item_discrimination, diff):
    mesh = plsc.VectorSubcoreMesh(core_axis_name="c", subcore_axis_name="s")
    run = pl.kernel(
        _mirt_body,
        out_type=jax.ShapeDtypeStruct((B,), jnp.float32),
        mesh=mesh,
        scratch_types=[
            pltpu.VMEM((BPW,), jnp.int32),
            pltpu.VMEM((BPW,), jnp.int32),
            pltpu.VMEM((2, CH, D), jnp.float32),
            pltpu.VMEM((2, CH, D), jnp.float32),
            pltpu.VMEM((2, CH), jnp.float32),
            pltpu.VMEM((BPW,), jnp.float32),
            pltpu.VMEM((16 * 17,), jnp.float32),
            pltpu.SemaphoreType.DMA,
            pltpu.SemaphoreType.DMA,
            pltpu.SemaphoreType.DMA,
        ],
        compiler_params=pltpu.CompilerParams(needs_layout_passes=False),
    )
    return run(uids, iids, users_emb, item_discrimination, diff)


def kernel(user_ids, item_ids, concept_ids, users_emb, item_discrimination,
           item_difficulty):
    del concept_ids
    return _mirt_sc(user_ids, item_ids, users_emb, item_discrimination,
                    item_difficulty.reshape(-1))


# speedup vs baseline: 1.1000x; 1.0022x over previous
import jax
import jax.numpy as jnp
from jax import lax
from jax.experimental import pallas as pl
from jax.experimental.pallas import tpu as pltpu
from jax.experimental.pallas import tpu_sc as plsc

B = 16384
D = 128
NC = 2
NS = 16
NW = NC * NS
BPW = B // NW
CH = 128
NCH = BPW // CH
GROUPS = CH // 16


def _mirt_body(uids_hbm, iids_hbm, emb_hbm, disc_hbm, diff_hbm, out_hbm,
               uid_v, iid_v, theta_v, a_v, b_v, out_v, pscr_v,
               sem0, sem1, sem_out):
    wid = lax.axis_index("s") * NC + lax.axis_index("c")
    base = wid * BPW

    cp_u = pltpu.async_copy(uids_hbm.at[pl.ds(base, BPW)], uid_v, sem0)
    cp_i = pltpu.async_copy(iids_hbm.at[pl.ds(base, BPW)], iid_v, sem1)
    cp_u.wait()
    cp_i.wait()

    iota = lax.iota(jnp.int32, 16)
    sems = (sem0, sem1)

    def start(c):
        p = c % 2
        ids = pl.ds(c * CH, CH)
        return (
            pltpu.async_copy(emb_hbm.at[uid_v.at[ids]], theta_v.at[p], sems[p]),
            pltpu.async_copy(disc_hbm.at[iid_v.at[ids]], a_v.at[p], sems[p]),
            pltpu.async_copy(diff_hbm.at[iid_v.at[ids]], b_v.at[p], sems[p]),
        )

    pending = start(0)
    out_pending = []
    for c in range(NCH):
        p = c % 2
        for cp in pending:
            cp.wait()
        if c + 1 < NCH:
            pending = start(c + 1)

        def group_body(g, _, p=p, c=c):
            for eb in range(0, 16, 4):
                accs = []
                for e16 in range(eb, eb + 4):
                    e = g * 16 + e16
                    acc = None
                    for j in range(D // 16):
                        tv = theta_v[p, e, pl.ds(j * 16, 16)]
                        av = a_v[p, e, pl.ds(j * 16, 16)]
                        prod = tv * av
                        acc = prod if acc is None else acc + prod
                    accs.append(acc)
                for k, acc in enumerate(accs):
                    pscr_v[pl.ds((eb + k) * 17, 16)] = acc
            idx = iota * 17
            out16 = None
            for j in range(16):
                col = plsc.load_gather(pscr_v, [idx])
                out16 = col if out16 is None else out16 + col
                idx = idx + 1
            b16 = b_v[p, pl.ds(g * 16, 16)]
            x = out16 - b16
            out_v[pl.ds(c * CH + g * 16, 16)] = 1.0 / (1.0 + jnp.exp(-x))
            return 0

        lax.fori_loop(0, GROUPS, group_body, 0)

        out_pending.append(pltpu.async_copy(
            out_v.at[pl.ds(c * CH, CH)],
            out_hbm.at[pl.ds(base + c * CH, CH)], sem_out))

    for cp in out_pending:
        cp.wait()


@jax.jit
def _mirt_sc(uids, iids, users_emb, item_discrimination, diff):
    mesh = plsc.VectorSubcoreMesh(core_axis_name="c", subcore_axis_name="s")
    run = pl.kernel(
        _mirt_body,
        out_type=jax.ShapeDtypeStruct((B,), jnp.float32),
        mesh=mesh,
        scratch_types=[
            pltpu.VMEM((BPW,), jnp.int32),
            pltpu.VMEM((BPW,), jnp.int32),
            pltpu.VMEM((2, CH, D), jnp.float32),
            pltpu.VMEM((2, CH, D), jnp.float32),
            pltpu.VMEM((2, CH), jnp.float32),
            pltpu.VMEM((BPW,), jnp.float32),
            pltpu.VMEM((16 * 17,), jnp.float32),
            pltpu.SemaphoreType.DMA,
            pltpu.SemaphoreType.DMA,
            pltpu.SemaphoreType.DMA,
        ],
        compiler_params=pltpu.CompilerParams(needs_layout_passes=False),
    )
    return run(uids, iids, users_emb, item_discrimination, diff)


def kernel(user_ids, item_ids, concept_ids, users_emb, item_discrimination,
           item_difficulty):
    del concept_ids
    return _mirt_sc(user_ids.astype(jnp.int32), item_ids.astype(jnp.int32),
                    users_emb, item_discrimination,
                    item_difficulty.reshape(-1))
